# baseline (speedup 1.0000x reference)
import jax
import jax.numpy as jnp
from jax.experimental import pallas as pl
from jax.experimental.pallas import tpu as pltpu

B, C_IN, HF, WF = 8, 64, 128, 128
NUM_CLASSES = 80
K = 100
HW = HF * WF
NROWS = NUM_CLASSES * HF
NEG = -1e30
BIGI = 2**30
G = 2
PB = B // G


def _det_kernel(x_ref, whm_ref, wrw_ref, out_ref, hm_scr, rw_all,
                cand_v, cand_g, mrow_scr, grow_scr, sel_smem):
    i = pl.program_id(1)

    @pl.when(i < PB)
    def _per_image():
        xb = x_ref[0]

        z = jnp.dot(whm_ref[...], xb, preferred_element_type=jnp.float32)
        rw = jnp.dot(wrw_ref[...], xb, preferred_element_type=jnp.float32)
        rw_all[pl.ds(i * 4 * HF, 4 * HF), :] = rw.reshape(4 * HF, WF)

        z3 = z.reshape(NUM_CLASSES, HF, WF)
        negw = jnp.full((NUM_CLASSES, HF, 1), NEG, jnp.float32)
        zl = jnp.concatenate([z3[:, :, 1:], negw], axis=2)
        zr = jnp.concatenate([negw, z3[:, :, :-1]], axis=2)
        mw = jnp.maximum(jnp.maximum(zl, zr), z3)
        negh = jnp.full((NUM_CLASSES, 1, WF), NEG, jnp.float32)
        mu = jnp.concatenate([mw[:, 1:, :], negh], axis=1)
        md = jnp.concatenate([negh, mw[:, :-1, :]], axis=1)
        hmax = jnp.maximum(jnp.maximum(mu, md), mw)
        znms = jnp.where(hmax == z3, z3, NEG)

        hm_scr[...] = znms.reshape(NROWS, WF)
        rowmax = jnp.max(znms, axis=2).reshape(NUM_CLASSES, HF)

        ridx = (jax.lax.broadcasted_iota(jnp.int32, (NUM_CLASSES, HF), 0) * HF
                + jax.lax.broadcasted_iota(jnp.int32, (NUM_CLASSES, HF), 1))
        col = jax.lax.broadcasted_iota(jnp.int32, (1, WF), 1)

        def body_a1(j, vals):
            m = jnp.max(vals)
            r = jnp.min(jnp.where(vals == m, ridx, BIGI))
            sel_smem[j] = r
            return jnp.where(ridx == r, -jnp.inf, vals)

        jax.lax.fori_loop(0, K, body_a1, rowmax, unroll=2)

        def body_a2(j, carry):
            r = sel_smem[j]
            cand_v[pl.ds(i * K + j, 1), :] = hm_scr[pl.ds(r, 1), :]
            cand_g[pl.ds(i * K + j, 1), :] = r * WF + col
            return carry

        jax.lax.fori_loop(0, K, body_a2, 0, unroll=4)

    @pl.when(i == PB)
    def _extract():
        lane = jax.lax.broadcasted_iota(jnp.int32, (1, WF), 1)

        def body_b(j, carry):
            for bb in range(PB):
                v = cand_v[bb * K:(bb + 1) * K, :]
                gi = cand_g[bb * K:(bb + 1) * K, :]
                m = jnp.max(v)
                g = jnp.min(jnp.where(v == m, gi, BIGI))
                cand_v[bb * K:(bb + 1) * K, :] = jnp.where(gi == g,
                                                           -jnp.inf, v)
                mrow_scr[pl.ds(bb * K + j, 1), :] = m + jnp.zeros((1, WF),
                                                                  jnp.float32)
                grow_scr[pl.ds(bb * K + j, 1), :] = g + jnp.zeros((1, WF),
                                                                  jnp.int32)
            return carry

        jax.lax.fori_loop(0, K, body_b, 0, unroll=False)

        lanek = jax.lax.broadcasted_iota(jnp.int32, (K, WF), 1)
        for bb in range(PB):
            g = grow_scr[bb * K:(bb + 1) * K, :]
            m = mrow_scr[bb * K:(bb + 1) * K, :]
            c = g // HW
            sp = g - c * HW
            yy = sp // WF
            xx = sp - yy * WF
            yhot = (lanek == yy).astype(jnp.float32)
            xhot = (lanek == xx).astype(jnp.float32)
            base = bb * 4 * HF
            vals = []
            for h in range(4):
                rows = jnp.dot(yhot, rw_all[base + h * HF:base + (h + 1) * HF,
                                            :],
                               preferred_element_type=jnp.float32)
                vals.append(jnp.sum(rows * xhot, axis=1, keepdims=True))
            regx, regy, ww, hh = vals
            score = jax.nn.sigmoid(m)
            xs = xx.astype(jnp.float32) + regx
            ys = yy.astype(jnp.float32) + regy
            row = ((lanek == 0) * (xs - ww * 0.5)
                   + (lanek == 1) * (ys - hh * 0.5)
                   + (lanek == 2) * (xs + ww * 0.5)
                   + (lanek == 3) * (ys + hh * 0.5)
                   + (lanek == 4) * score
                   + (lanek == 5) * c.astype(jnp.float32))
            out_ref[bb, :, :] = row[:, :6]


@jax.jit
def kernel(x, W_hm, W_wh, W_reg):
    xf = x.reshape(B, C_IN, HW)
    wrw = jnp.concatenate([W_reg, W_wh], axis=0)
    dets = pl.pallas_call(
        _det_kernel,
        grid=(G, PB + 1),
        in_specs=[
            pl.BlockSpec((1, C_IN, HW),
                         lambda c, i: (c * PB + jnp.minimum(i, PB - 1), 0, 0)),
            pl.BlockSpec((NUM_CLASSES, C_IN), lambda c, i: (0, 0)),
            pl.BlockSpec((4, C_IN), lambda c, i: (0, 0)),
        ],
        out_specs=pl.BlockSpec((PB, K, 6), lambda c, i: (c, 0, 0)),
        out_shape=jax.ShapeDtypeStruct((B, K, 6), jnp.float32),
        scratch_shapes=[
            pltpu.VMEM((NROWS, WF), jnp.float32),
            pltpu.VMEM((PB * 4 * HF, WF), jnp.float32),
            pltpu.VMEM((PB * K, WF), jnp.float32),
            pltpu.VMEM((PB * K, WF), jnp.int32),
            pltpu.VMEM((PB * K, WF), jnp.float32),
            pltpu.VMEM((PB * K, WF), jnp.int32),
            pltpu.SMEM((K,), jnp.int32),
        ],
        compiler_params=pltpu.CompilerParams(
            dimension_semantics=("parallel", "arbitrary"),
        ),
    )(xf, W_hm, wrw)
    return dets


# speedup vs baseline: 1.0001x; 1.0001x over previous
import jax
import jax.numpy as jnp
from jax.experimental import pallas as pl
from jax.experimental.pallas import tpu as pltpu

B, C_IN, HF, WF = 8, 64, 128, 128
NUM_CLASSES = 80
K = 100
HW = HF * WF
NROWS = NUM_CLASSES * HF
NEG = -1e30
BIGI = 2**30
G = 2
PB = B // G


def _det_kernel(x_ref, whm_ref, wrw_ref, out_ref, hm_scr, rw_all,
                cv0, cv1, cv2, cv3, cg0, cg1, cg2, cg3,
                mrow_scr, grow_scr, sel_smem):
    cand_v = [cv0, cv1, cv2, cv3]
    cand_g = [cg0, cg1, cg2, cg3]
    i = pl.program_id(1)

    @pl.when(i < PB)
    def _per_image():
        xb = x_ref[0]

        z = jnp.dot(whm_ref[...], xb, preferred_element_type=jnp.float32)
        rw = jnp.dot(wrw_ref[...], xb, preferred_element_type=jnp.float32)
        rw_all[pl.ds(i * 4 * HF, 4 * HF), :] = rw.reshape(4 * HF, WF)

        z3 = z.reshape(NUM_CLASSES, HF, WF)
        negw = jnp.full((NUM_CLASSES, HF, 1), NEG, jnp.float32)
        zl = jnp.concatenate([z3[:, :, 1:], negw], axis=2)
        zr = jnp.concatenate([negw, z3[:, :, :-1]], axis=2)
        mw = jnp.maximum(jnp.maximum(zl, zr), z3)
        negh = jnp.full((NUM_CLASSES, 1, WF), NEG, jnp.float32)
        mu = jnp.concatenate([mw[:, 1:, :], negh], axis=1)
        md = jnp.concatenate([negh, mw[:, :-1, :]], axis=1)
        hmax = jnp.maximum(jnp.maximum(mu, md), mw)
        znms = jnp.where(hmax == z3, z3, NEG)

        hm_scr[...] = znms.reshape(NROWS, WF)
        rowmax = jnp.max(znms, axis=2).reshape(NUM_CLASSES, HF)

        ridx = (jax.lax.broadcasted_iota(jnp.int32, (NUM_CLASSES, HF), 0) * HF
                + jax.lax.broadcasted_iota(jnp.int32, (NUM_CLASSES, HF), 1))
        col = jax.lax.broadcasted_iota(jnp.int32, (1, WF), 1)

        def body_a1(j, vals):
            m = jnp.max(vals)
            r = jnp.min(jnp.where(vals == m, ridx, BIGI))
            sel_smem[j] = r
            return jnp.where(ridx == r, -jnp.inf, vals)

        jax.lax.fori_loop(0, K, body_a1, rowmax, unroll=2)

        for bb in range(PB):
            @pl.when(i == bb)
            def _gather(bb=bb):
                def body_a2(j, carry):
                    r = sel_smem[j]
                    cand_v[bb][pl.ds(j, 1), :] = hm_scr[pl.ds(r, 1), :]
                    cand_g[bb][pl.ds(j, 1), :] = r * WF + col
                    return carry

                jax.lax.fori_loop(0, K, body_a2, 0, unroll=4)

    @pl.when(i == PB)
    def _extract():
        lane = jax.lax.broadcasted_iota(jnp.int32, (1, WF), 1)

        def body_b(j, carry):
            for bb in range(PB):
                v = cand_v[bb][...]
                gi = cand_g[bb][...]
                m = jnp.max(v)
                g = jnp.min(jnp.where(v == m, gi, BIGI))
                cand_v[bb][...] = jnp.where(gi == g, -jnp.inf, v)
                mrow_scr[pl.ds(bb * K + j, 1), :] = m + jnp.zeros((1, WF),
                                                                  jnp.float32)
                grow_scr[pl.ds(bb * K + j, 1), :] = g + jnp.zeros((1, WF),
                                                                  jnp.int32)
            return carry

        jax.lax.fori_loop(0, K, body_b, 0, unroll=False)

        lanek = jax.lax.broadcasted_iota(jnp.int32, (K, WF), 1)
        for bb in range(PB):
            g = grow_scr[bb * K:(bb + 1) * K, :]
            m = mrow_scr[bb * K:(bb + 1) * K, :]
            c = g // HW
            sp = g - c * HW
            yy = sp // WF
            xx = sp - yy * WF
            yhot = (lanek == yy).astype(jnp.float32)
            xhot = (lanek == xx).astype(jnp.float32)
            base = bb * 4 * HF
            vals = []
            for h in range(4):
                rows = jnp.dot(yhot, rw_all[base + h * HF:base + (h + 1) * HF,
                                            :],
                               preferred_element_type=jnp.float32)
                vals.append(jnp.sum(rows * xhot, axis=1, keepdims=True))
            regx, regy, ww, hh = vals
            score = jax.nn.sigmoid(m)
            xs = xx.astype(jnp.float32) + regx
            ys = yy.astype(jnp.float32) + regy
            row = ((lanek == 0) * (xs - ww * 0.5)
                   + (lanek == 1) * (ys - hh * 0.5)
                   + (lanek == 2) * (xs + ww * 0.5)
                   + (lanek == 3) * (ys + hh * 0.5)
                   + (lanek == 4) * score
                   + (lanek == 5) * c.astype(jnp.float32))
            out_ref[bb, :, :] = row[:, :6]


@jax.jit
def kernel(x, W_hm, W_wh, W_reg):
    xf = x.reshape(B, C_IN, HW)
    wrw = jnp.concatenate([W_reg, W_wh], axis=0)
    dets = pl.pallas_call(
        _det_kernel,
        grid=(G, PB + 1),
        in_specs=[
            pl.BlockSpec((1, C_IN, HW),
                         lambda c, i: (c * PB + jnp.minimum(i, PB - 1), 0, 0)),
            pl.BlockSpec((NUM_CLASSES, C_IN), lambda c, i: (0, 0)),
            pl.BlockSpec((4, C_IN), lambda c, i: (0, 0)),
        ],
        out_specs=pl.BlockSpec((PB, K, 6), lambda c, i: (c, 0, 0)),
        out_shape=jax.ShapeDtypeStruct((B, K, 6), jnp.float32),
        scratch_shapes=(
            [pltpu.VMEM((NROWS, WF), jnp.float32),
             pltpu.VMEM((PB * 4 * HF, WF), jnp.float32)]
            + [pltpu.VMEM((K, WF), jnp.float32) for _ in range(PB)]
            + [pltpu.VMEM((K, WF), jnp.int32) for _ in range(PB)]
            + [pltpu.VMEM((PB * K, WF), jnp.float32),
               pltpu.VMEM((PB * K, WF), jnp.int32),
               pltpu.SMEM((K,), jnp.int32)]
        ),
        compiler_params=pltpu.CompilerParams(
            dimension_semantics=("parallel", "arbitrary"),
        ),
    )(xf, W_hm, wrw)
    return dets
